# baseline (speedup 1.0000x reference)
import functools

import jax
import jax.numpy as jnp
from jax import lax
from jax.experimental import pallas as pl
from jax.experimental.pallas import tpu as pltpu
from jax.experimental.pallas import tpu_sc as plsc

NUM_TYPES = 21
EMB = 128
NUM_ATOMS = 524288
NC, NS = 2, 16
NW = NC * NS
CHUNK = 128
LANES = 16
ROWS_PER_W = NUM_ATOMS // NW
CHUNKS_PER_W = ROWS_PER_W // CHUNK
NBUF = 4
UNROLL = 4


def _make_gather():
  mesh = plsc.VectorSubcoreMesh(core_axis_name="c", subcore_axis_name="s")

  @functools.partial(
      pl.kernel,
      out_type=jax.ShapeDtypeStruct((NUM_ATOMS, EMB), jnp.float32),
      mesh=mesh,
      scratch_types=[
          pltpu.VMEM((NUM_TYPES, EMB), jnp.float32),
          pltpu.VMEM((ROWS_PER_W,), jnp.int32),
          pltpu.VMEM((NBUF, CHUNK, EMB), jnp.float32),
          pltpu.SemaphoreType.DMA,
          [pltpu.SemaphoreType.DMA] * NBUF,
      ],
  )
  def k(w_hbm, idx_hbm, out_hbm, w_loc, idx_v, rows, isem, wsems):
    cid = lax.axis_index("c")
    sid = lax.axis_index("s")
    wid = sid * NC + cid
    base = wid * ROWS_PER_W

    idx_copy = pltpu.make_async_copy(idx_hbm.at[pl.ds(base, ROWS_PER_W)],
                                     idx_v, isem)
    idx_copy.start()
    pltpu.sync_copy(w_hbm, w_loc)
    idx_copy.wait()

    def assemble(j, b):
      buf = rows.at[b]

      def rows_step(r16, carry):
        kv = idx_v[pl.ds(j * CHUNK + r16 * LANES, LANES)]
        nslice = EMB // LANES

        def pair_srcs(u):
          return w_loc.at[kv[u]], w_loc.at[kv[u + 1]]

        def load_one(srcs_u, c):
          return srcs_u[c // nslice][pl.ds((c % nslice) * LANES, LANES)]

        def store_one(u, c, val):
          dst = buf.at[r16 * LANES + u + c // nslice]
          dst[pl.ds((c % nslice) * LANES, LANES)] = val

        s0 = pair_srcs(0)
        vals = [load_one(s0, c) for c in range(2 * nslice)]
        for u in range(2, LANES, 2):
          su = pair_srcs(u)
          new_vals = []
          for c in range(2 * nslice):
            new_vals.append(load_one(su, c))
            store_one(u - 2, c, vals[c])
          vals = new_vals
        for c in range(2 * nslice):
          store_one(LANES - 2, c, vals[c])
        return carry

      lax.fori_loop(0, CHUNK // LANES, rows_step, 0)

    def start_write(j, b):
      pltpu.make_async_copy(
          rows.at[b], out_hbm.at[pl.ds(base + j * CHUNK, CHUNK)],
          wsems[b]).start()

    def wait_write(b):
      pltpu.make_async_copy(
          rows.at[b], out_hbm.at[pl.ds(base, CHUNK)], wsems[b]).wait()

    def body(j4, carry):
      for b in range(NBUF):
        j = j4 * NBUF + b

        @pl.when(j >= NBUF)
        def _():
          wait_write(b)

        assemble(j, b)
        start_write(j, b)
      return carry

    lax.fori_loop(0, CHUNKS_PER_W // NBUF, body, 0)

    for b in range(NBUF):
      wait_write(b)

  return k


_gather = _make_gather()


def kernel(residue_indices, W):
  idx = residue_indices.astype(jnp.int32)
  return _gather(W, idx)


# speedup vs baseline: 1.2352x; 1.2352x over previous
import functools

import jax
import jax.numpy as jnp
from jax import lax
from jax.experimental import pallas as pl
from jax.experimental.pallas import tpu as pltpu
from jax.experimental.pallas import tpu_sc as plsc

NUM_TYPES = 21
EMB = 128
NUM_ATOMS = 524288
NC, NS = 2, 16
NW = NC * NS
CHUNK = 128
LANES = 16
NSL = EMB // LANES
ROWS_PER_W = NUM_ATOMS // NW
CHUNKS_PER_W = ROWS_PER_W // CHUNK
NBUF = 4


def _make_gather():
  mesh = plsc.VectorSubcoreMesh(core_axis_name="c", subcore_axis_name="s")

  @functools.partial(
      pl.kernel,
      out_type=jax.ShapeDtypeStruct((NUM_ATOMS, EMB), jnp.float32),
      mesh=mesh,
      scratch_types=[
          pltpu.VMEM_SHARED((NS * NUM_TYPES, EMB), jnp.float32),
          pltpu.VMEM((NUM_TYPES, EMB), jnp.float32),
          pltpu.VMEM((ROWS_PER_W,), jnp.int32),
          pltpu.VMEM((NBUF, CHUNK, EMB), jnp.float32),
          pltpu.SemaphoreType.DMA,
          [pltpu.SemaphoreType.DMA] * NBUF,
          [pltpu.SemaphoreType.DMA] * NBUF,
      ],
  )
  def k(w_hbm, idx_hbm, out_hbm, w_sh, w_loc, idx_v, rows, isem, gsems,
        wsems):
    cid = lax.axis_index("c")
    sid = lax.axis_index("s")
    wid = sid * NC + cid
    base = wid * ROWS_PER_W

    idx_copy = pltpu.make_async_copy(idx_hbm.at[pl.ds(base, ROWS_PER_W)],
                                     idx_v, isem)
    idx_copy.start()
    pltpu.sync_copy(w_hbm, w_loc)
    w_my = w_sh.at[pl.ds(sid * NUM_TYPES, NUM_TYPES)]
    pltpu.sync_copy(w_loc, w_my)
    idx_copy.wait()

    def start_gather(j, b):
      pltpu.async_copy(w_my.at[idx_v.at[pl.ds(j * CHUNK, CHUNK)]],
                       rows.at[b], gsems[b])

    def wait_gather(b):
      pltpu.make_async_copy(w_my.at[idx_v.at[pl.ds(0, CHUNK)]], rows.at[b],
                            gsems[b]).wait()

    def start_write(j, b):
      pltpu.make_async_copy(
          rows.at[b], out_hbm.at[pl.ds(base + j * CHUNK, CHUNK)],
          wsems[b]).start()

    def wait_write(b):
      pltpu.make_async_copy(
          rows.at[b], out_hbm.at[pl.ds(base, CHUNK)], wsems[b]).wait()

    def assemble(j, b):
      buf = rows.at[b]

      def rows_step(r16, carry):
        kv = idx_v[pl.ds(j * CHUNK + r16 * LANES, LANES)]

        def pair_srcs(u):
          return w_loc.at[kv[u]], w_loc.at[kv[u + 1]]

        def load_one(srcs_u, c):
          return srcs_u[c // NSL][pl.ds((c % NSL) * LANES, LANES)]

        def store_one(u, c, val):
          dst = buf.at[r16 * LANES + u + c // NSL]
          dst[pl.ds((c % NSL) * LANES, LANES)] = val

        s0 = pair_srcs(0)
        vals = [load_one(s0, c) for c in range(2 * NSL)]
        for u in range(2, LANES, 2):
          su = pair_srcs(u)
          new_vals = []
          for c in range(2 * NSL):
            new_vals.append(load_one(su, c))
            store_one(u - 2, c, vals[c])
          vals = new_vals
        for c in range(2 * NSL):
          store_one(LANES - 2, c, vals[c])
        return carry

      lax.fori_loop(0, CHUNK // LANES, rows_step, 0)

    start_gather(0, 0)

    def body(j4, carry):
      for b in range(NBUF):
        j = j4 * NBUF + b
        if b % 2 == 0:
          wait_gather(b)
          start_write(j, b)
          bn = (b + 2) % NBUF

          @pl.when(j + 2 < CHUNKS_PER_W)
          def _():
            @pl.when(j >= 2)
            def _():
              wait_write(bn)

            start_gather(j + 2, bn)
        else:
          @pl.when(j >= NBUF)
          def _():
            wait_write(b)

          assemble(j, b)
          start_write(j, b)
      return carry

    lax.fori_loop(0, CHUNKS_PER_W // NBUF, body, 0)

    for b in range(NBUF):
      wait_write(b)

  return k


_gather = _make_gather()


def kernel(residue_indices, W):
  idx = residue_indices.astype(jnp.int32)
  return _gather(W, idx)


# speedup vs baseline: 1.2380x; 1.0022x over previous
import functools

import jax
import jax.numpy as jnp
from jax import lax
from jax.experimental import pallas as pl
from jax.experimental.pallas import tpu as pltpu
from jax.experimental.pallas import tpu_sc as plsc

NUM_TYPES = 21
EMB = 128
NUM_ATOMS = 524288
NC, NS = 2, 16
NW = NC * NS
CHUNK = 128
LANES = 16
NSL = EMB // LANES
ROWS_PER_W = NUM_ATOMS // NW
CHUNKS_PER_W = ROWS_PER_W // CHUNK
NBUF = 4


def _make_gather():
  mesh = plsc.VectorSubcoreMesh(core_axis_name="c", subcore_axis_name="s")

  @functools.partial(
      pl.kernel,
      out_type=jax.ShapeDtypeStruct((NUM_ATOMS, EMB), jnp.float32),
      mesh=mesh,
      scratch_types=[
          pltpu.VMEM_SHARED((NS * NUM_TYPES, EMB), jnp.float32),
          pltpu.VMEM((NUM_TYPES, EMB), jnp.float32),
          pltpu.VMEM((CHUNKS_PER_W, CHUNK), jnp.int32),
          pltpu.VMEM((NBUF, CHUNK, EMB), jnp.float32),
          pltpu.SemaphoreType.DMA,
          [pltpu.SemaphoreType.DMA] * NBUF,
          [pltpu.SemaphoreType.DMA] * NBUF,
      ],
  )
  def k(w_hbm, idx_hbm, out_hbm, w_sh, w_loc, idx_v, rows, isem, gsems,
        wsems):
    cid = lax.axis_index("c")
    sid = lax.axis_index("s")
    wid = sid * NC + cid
    base = wid * ROWS_PER_W

    idx_copy = pltpu.make_async_copy(
        idx_hbm.at[pl.ds(wid * CHUNKS_PER_W, CHUNKS_PER_W)], idx_v, isem)
    idx_copy.start()
    pltpu.sync_copy(w_hbm, w_loc)
    w_my = w_sh.at[pl.ds(sid * NUM_TYPES, NUM_TYPES)]
    pltpu.sync_copy(w_loc, w_my)
    idx_copy.wait()

    def start_gather(j, b):
      pltpu.async_copy(w_my.at[idx_v.at[j]], rows.at[b], gsems[b])

    def wait_gather(b):
      pltpu.make_async_copy(w_my.at[idx_v.at[0]], rows.at[b],
                            gsems[b]).wait()

    def start_write(j, b):
      pltpu.make_async_copy(
          rows.at[b], out_hbm.at[pl.ds(base + j * CHUNK, CHUNK)],
          wsems[b]).start()

    def wait_write(b):
      pltpu.make_async_copy(
          rows.at[b], out_hbm.at[pl.ds(base, CHUNK)], wsems[b]).wait()

    def assemble(j, b):
      buf = rows.at[b]

      def rows_step(r16, carry):
        kv = idx_v.at[j][pl.ds(r16 * LANES, LANES)]

        def pair_srcs(u):
          return w_loc.at[kv[u]], w_loc.at[kv[u + 1]]

        def load_one(srcs_u, c):
          return srcs_u[c // NSL][pl.ds((c % NSL) * LANES, LANES)]

        def store_one(u, c, val):
          dst = buf.at[r16 * LANES + u + c // NSL]
          dst[pl.ds((c % NSL) * LANES, LANES)] = val

        s0 = pair_srcs(0)
        vals = [load_one(s0, c) for c in range(2 * NSL)]
        for u in range(2, LANES, 2):
          su = pair_srcs(u)
          new_vals = []
          for c in range(2 * NSL):
            new_vals.append(load_one(su, c))
            store_one(u - 2, c, vals[c])
          vals = new_vals
        for c in range(2 * NSL):
          store_one(LANES - 2, c, vals[c])
        return carry

      lax.fori_loop(0, CHUNK // LANES, rows_step, 0)

    start_gather(0, 0)

    def body(j4, carry):
      for b in range(NBUF):
        j = j4 * NBUF + b
        if b % 2 == 0:
          wait_gather(b)
          start_write(j, b)
          bn = (b + 2) % NBUF

          @pl.when(j + 2 < CHUNKS_PER_W)
          def _():
            @pl.when(j >= 2)
            def _():
              wait_write(bn)

            start_gather(j + 2, bn)
        else:
          @pl.when(j >= NBUF)
          def _():
            wait_write(b)

          assemble(j, b)
          start_write(j, b)
      return carry

    lax.fori_loop(0, CHUNKS_PER_W // NBUF, body, 0)

    for b in range(NBUF):
      wait_write(b)

  return k


_gather = _make_gather()


def kernel(residue_indices, W):
  idx = residue_indices.astype(jnp.int32).reshape(NUM_ATOMS // CHUNK, CHUNK)
  return _gather(W, idx)
